# Initial kernel scaffold; baseline (speedup 1.0000x reference)
import jax
import jax.numpy as jnp
from jax.experimental import pallas as pl


def kernel(z, edge_index, edge_type, rel_emb):
    raise NotImplementedError("write your pallas kernel here")


# speedup vs baseline: 3.9535x; 3.9535x over previous
import functools

import jax
import jax.numpy as jnp
from jax import lax
from jax.experimental import pallas as pl
from jax.experimental.pallas import tpu as pltpu
from jax.experimental.pallas import tpu_sc as plsc

N_NODES = 10000
N_EDGES = 320000
D = 128
NREL = 1000

NC = 2
NS = 16
NW = NC * NS
CHUNK = 80
EPT = N_EDGES // NW
CPT = EPT // CHUNK
NROWS = N_EDGES // CHUNK

_mesh = plsc.VectorSubcoreMesh(core_axis_name="c", subcore_axis_name="s")


@functools.partial(
    pl.kernel,
    mesh=_mesh,
    out_type=(
        jax.ShapeDtypeStruct((N_EDGES,), jnp.float32),
        jax.ShapeDtypeStruct((N_EDGES, D), jnp.float32),
        jax.ShapeDtypeStruct((N_EDGES, D), jnp.float32),
        jax.ShapeDtypeStruct((N_EDGES, D), jnp.float32),
    ),
    scratch_types=[
        pltpu.VMEM((CPT, CHUNK), jnp.int32),
        pltpu.VMEM((CPT, CHUNK), jnp.int32),
        pltpu.VMEM((CPT, CHUNK), jnp.int32),
        pltpu.VMEM((CHUNK, D), jnp.float32),
        pltpu.VMEM((CHUNK, D), jnp.float32),
        pltpu.VMEM((CHUNK, D), jnp.float32),
        pltpu.VMEM((CHUNK,), jnp.float32),
        pltpu.SemaphoreType.DMA,
        pltpu.SemaphoreType.DMA,
        pltpu.SemaphoreType.DMA,
    ],
)
def _distmult_sc(z_hbm, src_hbm, dst_hbm, typ_hbm, rel_hbm,
                 score_hbm, zsrc_hbm, relo_hbm, zdst_hbm,
                 src_idx, dst_idx, typ_idx, s_rows, d_rows, r_rows,
                 score_v, sem0, sem1, sem2):
    wid = lax.axis_index("s") * NC + lax.axis_index("c")

    pltpu.sync_copy(src_hbm.at[wid], src_idx)
    pltpu.sync_copy(dst_hbm.at[wid], dst_idx)
    pltpu.sync_copy(typ_hbm.at[wid], typ_idx)

    lanes = lax.iota(jnp.int32, 16)

    def chunk_body(c, carry):
        cp0 = pltpu.async_copy(z_hbm.at[src_idx.at[c]], s_rows, sem0)
        cp1 = pltpu.async_copy(z_hbm.at[dst_idx.at[c]], d_rows, sem1)
        cp2 = pltpu.async_copy(rel_hbm.at[typ_idx.at[c]], r_rows, sem2)
        cp0.wait()
        cp1.wait()
        cp2.wait()

        for eb in range(CHUNK // 16):
            def edge_body(i, svec, eb=eb):
                e = eb * 16 + i
                acc = jnp.zeros((16,), jnp.float32)
                for j in range(D // 16):
                    sv = s_rows[e, pl.ds(j * 16, 16)]
                    rv = r_rows[e, pl.ds(j * 16, 16)]
                    dv = d_rows[e, pl.ds(j * 16, 16)]
                    acc = acc + sv * rv * dv
                for st in (1, 2, 4, 8):
                    acc = acc + acc.at[lanes ^ st].get(mode="promise_in_bounds")
                return jnp.where(lanes == i, acc, svec)

            svec = lax.fori_loop(0, 16, edge_body, jnp.zeros((16,), jnp.float32))
            score_v[pl.ds(eb * 16, 16)] = svec

        base = wid * EPT + c * CHUNK
        pltpu.sync_copy(s_rows, zsrc_hbm.at[pl.ds(base, CHUNK)])
        pltpu.sync_copy(r_rows, relo_hbm.at[pl.ds(base, CHUNK)])
        pltpu.sync_copy(d_rows, zdst_hbm.at[pl.ds(base, CHUNK)])
        pltpu.sync_copy(score_v, score_hbm.at[pl.ds(base, CHUNK)])
        return carry

    lax.fori_loop(0, CPT, chunk_body, 0)


def kernel(z, edge_index, edge_type, rel_emb):
    src = edge_index[0].astype(jnp.int32).reshape(NW, CPT, CHUNK)
    dst = edge_index[1].astype(jnp.int32).reshape(NW, CPT, CHUNK)
    typ = edge_type.astype(jnp.int32).reshape(NW, CPT, CHUNK)
    score, z_src, rel, z_dst = _distmult_sc(z, src, dst, typ, rel_emb)
    return score, z_src, rel, z_dst


# speedup vs baseline: 5.8807x; 1.4875x over previous
import functools

import jax
import jax.numpy as jnp
from jax import lax
from jax.experimental import pallas as pl
from jax.experimental.pallas import tpu as pltpu
from jax.experimental.pallas import tpu_sc as plsc

N_NODES = 10000
N_EDGES = 320000
D = 128
NREL = 1000

NC = 2
NS = 16
NW = NC * NS
CHUNK = 80
EPT = N_EDGES // NW
CPT = EPT // CHUNK

_mesh = plsc.VectorSubcoreMesh(core_axis_name="c", subcore_axis_name="s")


@functools.partial(
    pl.kernel,
    mesh=_mesh,
    out_type=(
        jax.ShapeDtypeStruct((N_EDGES,), jnp.float32),
        jax.ShapeDtypeStruct((N_EDGES, D), jnp.float32),
        jax.ShapeDtypeStruct((N_EDGES, D), jnp.float32),
        jax.ShapeDtypeStruct((N_EDGES, D), jnp.float32),
    ),
    scratch_types=[
        pltpu.VMEM((CPT, CHUNK), jnp.int32),
        pltpu.VMEM((CPT, CHUNK), jnp.int32),
        pltpu.VMEM((CPT, CHUNK), jnp.int32),
        pltpu.VMEM((CHUNK, D), jnp.float32),
        pltpu.VMEM((CHUNK, D), jnp.float32),
        pltpu.VMEM((CHUNK, D), jnp.float32),
        pltpu.VMEM((CHUNK, D), jnp.float32),
        pltpu.VMEM((CHUNK, D), jnp.float32),
        pltpu.VMEM((CHUNK, D), jnp.float32),
        pltpu.VMEM((CHUNK,), jnp.float32),
        pltpu.VMEM((CHUNK,), jnp.float32),
        pltpu.SemaphoreType.DMA,
        pltpu.SemaphoreType.DMA,
        pltpu.SemaphoreType.DMA,
        pltpu.SemaphoreType.DMA,
    ],
)
def _distmult_sc(z_hbm, src_hbm, dst_hbm, typ_hbm, rel_hbm,
                 score_hbm, zsrc_hbm, relo_hbm, zdst_hbm,
                 src_idx, dst_idx, typ_idx,
                 s0, d0, r0, s1, d1, r1, sc0, sc1,
                 gsem0, gsem1, wsem0, wsem1):
    wid = lax.axis_index("s") * NC + lax.axis_index("c")

    bufs = ((s0, d0, r0), (s1, d1, r1))
    scs = (sc0, sc1)
    gsems = (gsem0, gsem1)
    wsems = (wsem0, wsem1)

    pltpu.sync_copy(src_hbm.at[wid], src_idx)
    pltpu.sync_copy(dst_hbm.at[wid], dst_idx)
    pltpu.sync_copy(typ_hbm.at[wid], typ_idx)

    lanes = lax.iota(jnp.int32, 16)

    def start_gathers(c, b):
        s_r, d_r, r_r = bufs[b]
        pltpu.async_copy(z_hbm.at[src_idx.at[c]], s_r, gsems[b])
        pltpu.async_copy(z_hbm.at[dst_idx.at[c]], d_r, gsems[b])
        pltpu.async_copy(rel_hbm.at[typ_idx.at[c]], r_r, gsems[b])

    def wait_gathers(b):
        s_r, d_r, r_r = bufs[b]
        pltpu.make_async_copy(z_hbm.at[pl.ds(0, CHUNK)], s_r, gsems[b]).wait()
        pltpu.make_async_copy(z_hbm.at[pl.ds(0, CHUNK)], d_r, gsems[b]).wait()
        pltpu.make_async_copy(rel_hbm.at[pl.ds(0, CHUNK)], r_r, gsems[b]).wait()

    def start_writes(c, b):
        s_r, d_r, r_r = bufs[b]
        base = wid * EPT + c * CHUNK
        pltpu.async_copy(s_r, zsrc_hbm.at[pl.ds(base, CHUNK)], wsems[b])
        pltpu.async_copy(r_r, relo_hbm.at[pl.ds(base, CHUNK)], wsems[b])
        pltpu.async_copy(d_r, zdst_hbm.at[pl.ds(base, CHUNK)], wsems[b])
        pltpu.async_copy(scs[b], score_hbm.at[pl.ds(base, CHUNK)], wsems[b])

    def wait_writes(b):
        s_r, d_r, r_r = bufs[b]
        pltpu.make_async_copy(s_r, zsrc_hbm.at[pl.ds(0, CHUNK)], wsems[b]).wait()
        pltpu.make_async_copy(r_r, relo_hbm.at[pl.ds(0, CHUNK)], wsems[b]).wait()
        pltpu.make_async_copy(d_r, zdst_hbm.at[pl.ds(0, CHUNK)], wsems[b]).wait()
        pltpu.make_async_copy(scs[b], score_hbm.at[pl.ds(0, CHUNK)], wsems[b]).wait()

    def compute(b):
        s_r, d_r, r_r = bufs[b]
        for eb in range(CHUNK // 16):
            def edge_body(i, svec, eb=eb):
                e = eb * 16 + i
                acc = jnp.zeros((16,), jnp.float32)
                for j in range(D // 16):
                    sv = s_r[e, pl.ds(j * 16, 16)]
                    rv = r_r[e, pl.ds(j * 16, 16)]
                    dv = d_r[e, pl.ds(j * 16, 16)]
                    acc = acc + sv * rv * dv
                for st in (1, 2, 4, 8):
                    acc = acc + acc.at[lanes ^ st].get(mode="promise_in_bounds")
                return jnp.where(lanes == i, acc, svec)

            svec = lax.fori_loop(0, 16, edge_body, jnp.zeros((16,), jnp.float32))
            scs[b][pl.ds(eb * 16, 16)] = svec

    def process(c, b, first=False, last=False):
        if not first:
            wait_writes(1 - b)
        if not last:
            start_gathers(c + 1, 1 - b)
        wait_gathers(b)
        compute(b)
        start_writes(c, b)

    start_gathers(0, 0)
    process(0, 0, first=True)

    def pair_body(k, carry):
        process(2 * k + 1, 1)
        process(2 * k + 2, 0)
        return carry

    lax.fori_loop(0, (CPT - 3) // 2, pair_body, 0)
    process(CPT - 2, 1)
    process(CPT - 1, 0, last=True)
    wait_writes(0)


def kernel(z, edge_index, edge_type, rel_emb):
    src = edge_index[0].astype(jnp.int32).reshape(NW, CPT, CHUNK)
    dst = edge_index[1].astype(jnp.int32).reshape(NW, CPT, CHUNK)
    typ = edge_type.astype(jnp.int32).reshape(NW, CPT, CHUNK)
    score, z_src, rel, z_dst = _distmult_sc(z, src, dst, typ, rel_emb)
    return score, z_src, rel, z_dst


# speedup vs baseline: 5.8815x; 1.0001x over previous
import functools

import jax
import jax.numpy as jnp
from jax import lax
from jax.experimental import pallas as pl
from jax.experimental.pallas import tpu as pltpu
from jax.experimental.pallas import tpu_sc as plsc

N_NODES = 10000
N_EDGES = 320000
D = 128
NREL = 1000

NC = 2
NS = 16
NW = NC * NS
CHUNK = 80
EPT = N_EDGES // NW
CPT = EPT // CHUNK

_mesh = plsc.VectorSubcoreMesh(core_axis_name="c", subcore_axis_name="s")


@functools.partial(
    pl.kernel,
    mesh=_mesh,
    out_type=(
        jax.ShapeDtypeStruct((N_EDGES,), jnp.float32),
        jax.ShapeDtypeStruct((N_EDGES, D), jnp.float32),
        jax.ShapeDtypeStruct((N_EDGES, D), jnp.float32),
        jax.ShapeDtypeStruct((N_EDGES, D), jnp.float32),
    ),
    scratch_types=[
        pltpu.VMEM((CPT, CHUNK), jnp.int32),
        pltpu.VMEM((CPT, CHUNK), jnp.int32),
        pltpu.VMEM((CPT, CHUNK), jnp.int32),
        pltpu.VMEM((CHUNK, D), jnp.float32),
        pltpu.VMEM((CHUNK, D), jnp.float32),
        pltpu.VMEM((CHUNK, D), jnp.float32),
        pltpu.VMEM((CHUNK, D), jnp.float32),
        pltpu.VMEM((CHUNK, D), jnp.float32),
        pltpu.VMEM((CHUNK, D), jnp.float32),
        pltpu.VMEM((CHUNK,), jnp.float32),
        pltpu.VMEM((CHUNK,), jnp.float32),
        pltpu.SemaphoreType.DMA,
        pltpu.SemaphoreType.DMA,
        pltpu.SemaphoreType.DMA,
        pltpu.SemaphoreType.DMA,
    ],
)
def _distmult_sc(z_hbm, src_hbm, dst_hbm, typ_hbm, rel_hbm,
                 score_hbm, zsrc_hbm, relo_hbm, zdst_hbm,
                 src_idx, dst_idx, typ_idx,
                 s0, d0, r0, s1, d1, r1, sc0, sc1,
                 gsem0, gsem1, wsem0, wsem1):
    sid = lax.axis_index("s")
    wid = sid * NC + lax.axis_index("c")

    bufs = ((s0, d0, r0), (s1, d1, r1))
    scs = (sc0, sc1)
    gsems = (gsem0, gsem1)
    wsems = (wsem0, wsem1)

    pltpu.sync_copy(src_hbm.at[wid], src_idx)
    pltpu.sync_copy(dst_hbm.at[wid], dst_idx)
    pltpu.sync_copy(typ_hbm.at[wid], typ_idx)

    lanes = lax.iota(jnp.int32, 16)

    def start_gathers(c, b):
        s_r, d_r, r_r = bufs[b]
        pltpu.async_copy(z_hbm.at[src_idx.at[c]], s_r, gsems[b])
        pltpu.async_copy(z_hbm.at[dst_idx.at[c]], d_r, gsems[b])
        pltpu.async_copy(rel_hbm.at[typ_idx.at[c]], r_r, gsems[b])

    def wait_gathers(b):
        s_r, d_r, r_r = bufs[b]
        pltpu.make_async_copy(z_hbm.at[pl.ds(0, CHUNK)], s_r, gsems[b]).wait()
        pltpu.make_async_copy(z_hbm.at[pl.ds(0, CHUNK)], d_r, gsems[b]).wait()
        pltpu.make_async_copy(rel_hbm.at[pl.ds(0, CHUNK)], r_r, gsems[b]).wait()

    def start_writes(c, b):
        s_r, d_r, r_r = bufs[b]
        base = wid * EPT + c * CHUNK
        pltpu.async_copy(s_r, zsrc_hbm.at[pl.ds(base, CHUNK)], wsems[b])
        pltpu.async_copy(r_r, relo_hbm.at[pl.ds(base, CHUNK)], wsems[b])
        pltpu.async_copy(d_r, zdst_hbm.at[pl.ds(base, CHUNK)], wsems[b])
        pltpu.async_copy(scs[b], score_hbm.at[pl.ds(base, CHUNK)], wsems[b])

    def wait_writes(b):
        s_r, d_r, r_r = bufs[b]
        pltpu.make_async_copy(s_r, zsrc_hbm.at[pl.ds(0, CHUNK)], wsems[b]).wait()
        pltpu.make_async_copy(r_r, relo_hbm.at[pl.ds(0, CHUNK)], wsems[b]).wait()
        pltpu.make_async_copy(d_r, zdst_hbm.at[pl.ds(0, CHUNK)], wsems[b]).wait()
        pltpu.make_async_copy(scs[b], score_hbm.at[pl.ds(0, CHUNK)], wsems[b]).wait()

    def compute(b):
        s_r, d_r, r_r = bufs[b]
        for eb in range(CHUNK // 16):
            def edge_body(i, svec, eb=eb):
                e = eb * 16 + i
                acc = jnp.zeros((16,), jnp.float32)
                for j in range(D // 16):
                    sv = s_r[e, pl.ds(j * 16, 16)]
                    rv = r_r[e, pl.ds(j * 16, 16)]
                    dv = d_r[e, pl.ds(j * 16, 16)]
                    acc = acc + sv * rv * dv
                for st in (1, 2, 4, 8):
                    acc = acc + acc.at[lanes ^ st].get(mode="promise_in_bounds")
                return jnp.where(lanes == i, acc, svec)

            svec = lax.fori_loop(0, 16, edge_body, jnp.zeros((16,), jnp.float32))
            scs[b][pl.ds(eb * 16, 16)] = svec

    def process(c, b, first=False, last=False):
        if not first:
            wait_writes(1 - b)
        if not last:
            start_gathers(c + 1, 1 - b)
        wait_gathers(b)
        compute(b)
        start_writes(c, b)

    start_gathers(0, 0)
    process(0, 0, first=True)

    def pair_body(k, carry):
        process(2 * k + 1, 1)
        process(2 * k + 2, 0)
        return carry

    lax.fori_loop(0, (CPT - 3) // 2, pair_body, 0)
    process(CPT - 2, 1)
    process(CPT - 1, 0, last=True)
    wait_writes(0)


def kernel(z, edge_index, edge_type, rel_emb):
    src = edge_index[0].astype(jnp.int32).reshape(NW, CPT, CHUNK)
    dst = edge_index[1].astype(jnp.int32).reshape(NW, CPT, CHUNK)
    typ = edge_type.astype(jnp.int32).reshape(NW, CPT, CHUNK)
    score, z_src, rel, z_dst = _distmult_sc(z, src, dst, typ, rel_emb)
    return score, z_src, rel, z_dst
